# baseline (device time: 23203 ns/iter reference)
import jax
import jax.numpy as jnp
from jax import lax
from jax.experimental import pallas as pl
from jax.experimental.pallas import tpu as pltpu


def kernel(x, dy):
    m, d = x.shape
    _, f = dy.shape
    d_half = d // 2
    f_half = f // 2

    def body(x_ref, dy_ref, out_ref, p1s_ref, p1r_ref, p2s_ref, p2r_ref, sems):
        my_x = lax.axis_index("x")
        my_y = lax.axis_index("y")
        other_x = 1 - my_x
        other_y = 1 - my_y

        barrier = pltpu.get_barrier_semaphore()
        pl.semaphore_signal(barrier, inc=1, device_id=(my_x, other_y),
                            device_id_type=pl.DeviceIdType.MESH)
        pl.semaphore_signal(barrier, inc=1, device_id=(other_x, my_y),
                            device_id_type=pl.DeviceIdType.MESH)
        pl.semaphore_wait(barrier, 2)

        dy_bf = dy_ref[:, pl.ds(my_x * f_half, f_half)].astype(jnp.bfloat16)

        dims = (((0,), (0,)), ((), ()))

        x_send = x_ref[:, pl.ds(other_y * d_half, d_half)].astype(jnp.bfloat16)
        p1s_ref[...] = lax.dot_general(
            x_send, dy_bf, dims, preferred_element_type=jnp.float32
        ).astype(jnp.bfloat16)

        rdma1 = pltpu.make_async_remote_copy(
            src_ref=p1s_ref, dst_ref=p1r_ref,
            send_sem=sems.at[0], recv_sem=sems.at[1],
            device_id=(my_x, other_y), device_id_type=pl.DeviceIdType.MESH,
        )
        rdma1.start()

        x_keep = x_ref[:, pl.ds(my_y * d_half, d_half)].astype(jnp.bfloat16)
        partial_keep = lax.dot_general(
            x_keep, dy_bf, dims, preferred_element_type=jnp.float32
        )

        rdma1.wait()
        reduced = partial_keep + p1r_ref[...].astype(jnp.float32)

        p2s_ref[...] = reduced.astype(jnp.bfloat16)
        rdma2 = pltpu.make_async_remote_copy(
            src_ref=p2s_ref, dst_ref=p2r_ref,
            send_sem=sems.at[2], recv_sem=sems.at[3],
            device_id=(other_x, my_y), device_id_type=pl.DeviceIdType.MESH,
        )
        rdma2.start()

        out_ref[:, pl.ds(my_x * f_half, f_half)] = reduced
        rdma2.wait()
        out_ref[:, pl.ds(other_x * f_half, f_half)] = (
            p2r_ref[...].astype(jnp.float32)
        )

    return pl.pallas_call(
        body,
        out_shape=jax.ShapeDtypeStruct((d_half, f), jnp.float32),
        in_specs=[
            pl.BlockSpec(memory_space=pltpu.VMEM),
            pl.BlockSpec(memory_space=pltpu.VMEM),
        ],
        out_specs=pl.BlockSpec(memory_space=pltpu.VMEM),
        scratch_shapes=[
            pltpu.VMEM((d_half, f_half), jnp.bfloat16),
            pltpu.VMEM((d_half, f_half), jnp.bfloat16),
            pltpu.VMEM((d_half, f_half), jnp.bfloat16),
            pltpu.VMEM((d_half, f_half), jnp.bfloat16),
            pltpu.SemaphoreType.DMA((4,)),
        ],
        compiler_params=pltpu.CompilerParams(collective_id=0),
    )(x, dy)


# device time: 18706 ns/iter; 1.2404x vs baseline; 1.2404x over previous
import jax
import jax.numpy as jnp
from jax import lax
from jax.experimental import pallas as pl
from jax.experimental.pallas import tpu as pltpu

NC = 4


def kernel(x, dy):
    m, d = x.shape
    _, f = dy.shape
    d_half = d // 2
    f_half = f // 2
    cw = f_half // NC

    def body(x_ref, dy_ref, out_ref, dy_bf_ref,
             p1s_ref, p1r_ref, p2s_ref, p2r_ref, s1, r1, s2, r2):
        my_x = lax.axis_index("x")
        my_y = lax.axis_index("y")
        other_x = 1 - my_x
        other_y = 1 - my_y

        barrier = pltpu.get_barrier_semaphore()
        pl.semaphore_signal(barrier, inc=1, device_id=(my_x, other_y),
                            device_id_type=pl.DeviceIdType.MESH)
        pl.semaphore_signal(barrier, inc=1, device_id=(other_x, my_y),
                            device_id_type=pl.DeviceIdType.MESH)
        pl.semaphore_wait(barrier, 2)

        dy_bf_ref[...] = dy_ref[:, pl.ds(my_x * f_half, f_half)].astype(
            jnp.bfloat16
        )
        x_send = x_ref[:, pl.ds(other_y * d_half, d_half)].astype(jnp.bfloat16)
        x_keep = x_ref[:, pl.ds(my_y * d_half, d_half)].astype(jnp.bfloat16)

        dims = (((0,), (0,)), ((), ()))

        rdma1 = []
        for c in range(NC):
            p1s_ref[c] = lax.dot_general(
                x_send, dy_bf_ref[:, c * cw:(c + 1) * cw], dims,
                preferred_element_type=jnp.float32,
            ).astype(jnp.bfloat16)
            r = pltpu.make_async_remote_copy(
                src_ref=p1s_ref.at[c], dst_ref=p1r_ref.at[c],
                send_sem=s1.at[c], recv_sem=r1.at[c],
                device_id=(my_x, other_y),
                device_id_type=pl.DeviceIdType.MESH,
            )
            r.start()
            rdma1.append(r)

        rdma2 = []
        for c in range(NC):
            pk = lax.dot_general(
                x_keep, dy_bf_ref[:, c * cw:(c + 1) * cw], dims,
                preferred_element_type=jnp.float32,
            )
            rdma1[c].wait_recv()
            red = pk + p1r_ref[c].astype(jnp.float32)
            p2s_ref[c] = red.astype(jnp.bfloat16)
            r = pltpu.make_async_remote_copy(
                src_ref=p2s_ref.at[c], dst_ref=p2r_ref.at[c],
                send_sem=s2.at[c], recv_sem=r2.at[c],
                device_id=(other_x, my_y),
                device_id_type=pl.DeviceIdType.MESH,
            )
            r.start()
            rdma2.append(r)
            out_ref[:, pl.ds(my_x * f_half + c * cw, cw)] = red

        for c in range(NC):
            rdma2[c].wait_recv()
            out_ref[:, pl.ds(other_x * f_half + c * cw, cw)] = (
                p2r_ref[c].astype(jnp.float32)
            )
        for c in range(NC):
            rdma1[c].wait_send()
            rdma2[c].wait_send()

    return pl.pallas_call(
        body,
        out_shape=jax.ShapeDtypeStruct((d_half, f), jnp.float32),
        in_specs=[
            pl.BlockSpec(memory_space=pltpu.VMEM),
            pl.BlockSpec(memory_space=pltpu.VMEM),
        ],
        out_specs=pl.BlockSpec(memory_space=pltpu.VMEM),
        scratch_shapes=[
            pltpu.VMEM((m, f_half), jnp.bfloat16),
            pltpu.VMEM((NC, d_half, cw), jnp.bfloat16),
            pltpu.VMEM((NC, d_half, cw), jnp.bfloat16),
            pltpu.VMEM((NC, d_half, cw), jnp.bfloat16),
            pltpu.VMEM((NC, d_half, cw), jnp.bfloat16),
            pltpu.SemaphoreType.DMA((NC,)),
            pltpu.SemaphoreType.DMA((NC,)),
            pltpu.SemaphoreType.DMA((NC,)),
            pltpu.SemaphoreType.DMA((NC,)),
        ],
        compiler_params=pltpu.CompilerParams(collective_id=0),
    )(x, dy)


# device time: 18412 ns/iter; 1.2602x vs baseline; 1.0160x over previous
import jax
import jax.numpy as jnp
from jax import lax
from jax.experimental import pallas as pl
from jax.experimental.pallas import tpu as pltpu

NC = 4


def kernel(x, dy):
    m, d = x.shape
    _, f = dy.shape
    d_half = d // 2
    f_half = f // 2
    cw = f_half // NC

    def body(x_ref, dy_ref, out_ref, dy_bf_ref, p1s_ref, p1r_ref,
             s1, r1, s2, r2):
        my_x = lax.axis_index("x")
        my_y = lax.axis_index("y")
        other_x = 1 - my_x
        other_y = 1 - my_y

        barrier = pltpu.get_barrier_semaphore()
        pl.semaphore_signal(barrier, inc=1, device_id=(my_x, other_y),
                            device_id_type=pl.DeviceIdType.MESH)
        pl.semaphore_signal(barrier, inc=1, device_id=(other_x, my_y),
                            device_id_type=pl.DeviceIdType.MESH)
        pl.semaphore_wait(barrier, 2)

        dy_bf_ref[...] = dy_ref[:, pl.ds(my_x * f_half, f_half)].astype(
            jnp.bfloat16
        )
        x_send = x_ref[:, pl.ds(other_y * d_half, d_half)].astype(jnp.bfloat16)
        x_keep = x_ref[:, pl.ds(my_y * d_half, d_half)].astype(jnp.bfloat16)

        dims = (((0,), (0,)), ((), ()))

        rdma1 = []
        for c in range(NC):
            p1s_ref[c] = lax.dot_general(
                x_send, dy_bf_ref[:, c * cw:(c + 1) * cw], dims,
                preferred_element_type=jnp.float32,
            ).astype(jnp.bfloat16)
            r = pltpu.make_async_remote_copy(
                src_ref=p1s_ref.at[c], dst_ref=p1r_ref.at[c],
                send_sem=s1.at[c], recv_sem=r1.at[c],
                device_id=(my_x, other_y),
                device_id_type=pl.DeviceIdType.MESH,
            )
            r.start()
            rdma1.append(r)

        rdma2 = []
        for c in range(NC):
            pk = lax.dot_general(
                x_keep, dy_bf_ref[:, c * cw:(c + 1) * cw], dims,
                preferred_element_type=jnp.float32,
            )
            rdma1[c].wait_recv()
            red = pk + p1r_ref[c].astype(jnp.float32)
            out_ref[:, pl.ds(my_x * f_half + c * cw, cw)] = red.astype(
                jnp.bfloat16
            )
            r = pltpu.make_async_remote_copy(
                src_ref=out_ref.at[:, pl.ds(my_x * f_half + c * cw, cw)],
                dst_ref=out_ref.at[:, pl.ds(my_x * f_half + c * cw, cw)],
                send_sem=s2.at[c], recv_sem=r2.at[c],
                device_id=(other_x, my_y),
                device_id_type=pl.DeviceIdType.MESH,
            )
            r.start()
            rdma2.append(r)

        for c in range(NC):
            rdma2[c].wait_recv()
        for c in range(NC):
            rdma1[c].wait_send()
            rdma2[c].wait_send()

    return pl.pallas_call(
        body,
        out_shape=jax.ShapeDtypeStruct((d_half, f), jnp.bfloat16),
        in_specs=[
            pl.BlockSpec(memory_space=pltpu.VMEM),
            pl.BlockSpec(memory_space=pltpu.VMEM),
        ],
        out_specs=pl.BlockSpec(memory_space=pltpu.VMEM),
        scratch_shapes=[
            pltpu.VMEM((m, f_half), jnp.bfloat16),
            pltpu.VMEM((NC, d_half, cw), jnp.bfloat16),
            pltpu.VMEM((NC, d_half, cw), jnp.bfloat16),
            pltpu.SemaphoreType.DMA((NC,)),
            pltpu.SemaphoreType.DMA((NC,)),
            pltpu.SemaphoreType.DMA((NC,)),
            pltpu.SemaphoreType.DMA((NC,)),
        ],
        compiler_params=pltpu.CompilerParams(collective_id=0),
    )(x, dy)
